# baseline (speedup 1.0000x reference)
import functools

import jax
import jax.numpy as jnp
from jax import lax
from jax.experimental import pallas as pl
from jax.experimental.pallas import tpu as pltpu
from jax.experimental.pallas import tpu_sc as plsc

F32 = jnp.float32
NC = 2
NS = 16
KB = 64
PAD_ROWS = 32


def _edge_pad(e):
  per_tile = -(-e // (NC * NS * 4 * KB)) * 4 * KB
  return per_tile * NC * NS


def _node_pad(n):
  npad = -(-n // 128) * 128
  if npad - n < PAD_ROWS:
    npad += 128
  return npad


def _make_deg_kernel(npad, ep):
  et = ep // (NC * NS)
  nb = et // KB
  ng = nb // 4
  rpt = npad // NS

  mesh = plsc.VectorSubcoreMesh(
      core_axis_name="c", subcore_axis_name="s", num_cores=NC, num_subcores=NS)

  @functools.partial(
      pl.kernel,
      out_type=jax.ShapeDtypeStruct((NC, npad, 128), F32),
      mesh=mesh,
      scratch_types=[
          pltpu.VMEM((et,), jnp.int32),
          pltpu.VMEM((KB,), jnp.int32),
          pltpu.VMEM((KB,), jnp.int32),
          pltpu.VMEM((KB,), jnp.int32),
          pltpu.VMEM((KB,), jnp.int32),
          pltpu.VMEM((KB, 128), F32),
          pltpu.VMEM_SHARED((npad, 128), F32),
          pltpu.SemaphoreType.DMA,
          pltpu.SemaphoreType.DMA,
          pltpu.SemaphoreType.DMA,
          pltpu.SemaphoreType.DMA,
      ],
  )
  def deg_kernel(dst_hbm, ones_hbm, zeros_hbm, out_hbm, dst_t, sidx0, sidx1,
                 sidx2, sidx3, ones_t, acc, sem0, sem1, sem2, sem3):
    ci = lax.axis_index("c")
    si = lax.axis_index("s")
    t = ci * NS + si
    r0 = si * rpt
    sidx = (sidx0, sidx1, sidx2, sidx3)
    sems = (sem0, sem1, sem2, sem3)
    pltpu.sync_copy(dst_hbm.at[pl.ds(t * et, et)], dst_t)
    pltpu.sync_copy(ones_hbm, ones_t)
    pltpu.sync_copy(zeros_hbm.at[pl.ds(r0, rpt)], acc.at[pl.ds(r0, rpt)])
    plsc.subcore_barrier()

    def quad(g, carry):
      @pl.when(g > 0)
      def _():
        for i in range(4):
          pltpu.make_async_copy(ones_t, acc.at[sidx[i]], sems[i]).wait()

      for i in range(4):
        base = pl.multiple_of((4 * g + i) * KB, KB)
        for j in range(KB // 16):
          sidx[i][pl.ds(j * 16, 16)] = dst_t[pl.ds(base + j * 16, 16)]
        pltpu.async_copy(ones_t, acc.at[sidx[i]], sems[i], add=True)
      return carry

    lax.fori_loop(0, ng, quad, 0)
    for i in range(4):
      pltpu.make_async_copy(ones_t, acc.at[sidx[i]], sems[i]).wait()
    plsc.subcore_barrier()
    pltpu.sync_copy(acc.at[pl.ds(r0, rpt)],
                    out_hbm.at[ci, pl.ds(r0, rpt)])

  return deg_kernel


def _make_agg_kernel(npad, ep, f):
  fc = min(f, 128)
  cb = f // fc
  et = ep // (NC * NS)
  nb = et // KB
  rpt = npad // NS

  mesh = plsc.VectorSubcoreMesh(
      core_axis_name="c", subcore_axis_name="s", num_cores=NC, num_subcores=NS)

  @functools.partial(
      pl.kernel,
      out_type=jax.ShapeDtypeStruct((NC, npad, f), F32),
      mesh=mesh,
      scratch_types=[
          pltpu.VMEM((et,), jnp.int32),
          pltpu.VMEM((et,), jnp.int32),
          pltpu.VMEM((KB,), jnp.int32),
          pltpu.VMEM((KB,), jnp.int32),
          pltpu.VMEM((KB,), jnp.int32),
          pltpu.VMEM((KB,), jnp.int32),
          pltpu.VMEM((KB, fc), F32),
          pltpu.VMEM((KB, fc), F32),
          pltpu.VMEM((KB, fc), F32),
          pltpu.VMEM((KB, fc), F32),
          pltpu.VMEM_SHARED((npad, fc), F32),
          pltpu.SemaphoreType.DMA,
          pltpu.SemaphoreType.DMA,
          pltpu.SemaphoreType.DMA,
          pltpu.SemaphoreType.DMA,
          pltpu.SemaphoreType.DMA,
          pltpu.SemaphoreType.DMA,
          pltpu.SemaphoreType.DMA,
          pltpu.SemaphoreType.DMA,
      ],
  )
  def agg_kernel(table_hbm, src_hbm, dst_hbm, zeros_hbm, u_hbm, src_t, dst_t,
                 si0, si1, si2, si3, gb0, gb1, gb2, gb3, acc, gs0, gs1, gs2,
                 gs3, ss0, ss1, ss2, ss3):
    ci = lax.axis_index("c")
    si = lax.axis_index("s")
    t = ci * NS + si
    r0 = si * rpt
    sidx = (si0, si1, si2, si3)
    gbuf = (gb0, gb1, gb2, gb3)
    gsem = (gs0, gs1, gs2, gs3)
    ssem = (ss0, ss1, ss2, ss3)
    pltpu.sync_copy(src_hbm.at[pl.ds(t * et, et)], src_t)
    pltpu.sync_copy(dst_hbm.at[pl.ds(t * et, et)], dst_t)
    ng = nb // 4

    for c in range(cb):
      pltpu.sync_copy(zeros_hbm.at[pl.ds(r0, rpt)],
                      acc.at[pl.ds(r0, rpt)])
      plsc.subcore_barrier()

      def gather_src(base):
        gi = src_t.at[pl.ds(base, KB)]
        if cb > 1:
          return table_hbm.at[gi, pl.ds(c * fc, fc)]
        return table_hbm.at[gi]

      def quad(g, carry):
        @pl.when(g > 0)
        def _():
          for i in range(4):
            pltpu.make_async_copy(gbuf[i], acc.at[sidx[i]], ssem[i]).wait()

        for i in range(4):
          base = pl.multiple_of((4 * g + i) * KB, KB)
          for j in range(KB // 16):
            sidx[i][pl.ds(j * 16, 16)] = dst_t[pl.ds(base + j * 16, 16)]
          pltpu.async_copy(gather_src(base), gbuf[i], gsem[i])

        for i in range(4):
          base = pl.multiple_of((4 * g + i) * KB, KB)
          pltpu.make_async_copy(gather_src(base), gbuf[i], gsem[i]).wait()
          pltpu.async_copy(gbuf[i], acc.at[sidx[i]], ssem[i], add=True)
        return carry

      lax.fori_loop(0, ng, quad, 0)
      for i in range(4):
        pltpu.make_async_copy(gbuf[i], acc.at[sidx[i]], ssem[i]).wait()
      plsc.subcore_barrier()
      pltpu.sync_copy(acc.at[pl.ds(r0, rpt)],
                      u_hbm.at[ci, pl.ds(r0, rpt), pl.ds(c * fc, fc)])

  return agg_kernel


def _k1(x, w1, cnt2, mt=1000):
  n, kdim = x.shape
  fout = w1.shape[1]
  npad = cnt2.shape[1]

  def body(x_ref, w_ref, c_ref, o_ref, cs_ref):
    h = jnp.dot(x_ref[...], w_ref[...], preferred_element_type=F32)
    cs = c_ref[0] + c_ref[1]
    cs_ref[...] = cs
    norm = lax.rsqrt(cs[:, 0:1] + 1.0)
    o_ref[...] = h * norm

  return pl.pallas_call(
      body,
      grid=(n // mt,),
      in_specs=[
          pl.BlockSpec((mt, kdim), lambda i: (i, 0)),
          pl.BlockSpec((kdim, fout), lambda i: (0, 0)),
          pl.BlockSpec((NC, mt, 128), lambda i: (0, i, 0)),
      ],
      out_specs=[
          pl.BlockSpec((mt, fout), lambda i: (i, 0)),
          pl.BlockSpec((mt, 128), lambda i: (i, 0)),
      ],
      out_shape=[
          jax.ShapeDtypeStruct((n, fout), F32),
          jax.ShapeDtypeStruct((npad, 128), F32),
      ],
      compiler_params=pltpu.CompilerParams(
          dimension_semantics=("arbitrary",)),
  )(x, w1, cnt2)


def _k_mid(u, hp, cnt, b, w, mt=1000):
  n, fin = hp.shape
  fout = w.shape[1]

  def body(u_ref, h_ref, c_ref, b_ref, w_ref, o_ref):
    norm = lax.rsqrt(c_ref[:, 0:1] + 1.0)
    a = u_ref[0] + u_ref[1] + h_ref[...]
    a = jax.nn.relu(norm * a + b_ref[...])
    o_ref[...] = jnp.dot(a, w_ref[...], preferred_element_type=F32) * norm

  return pl.pallas_call(
      body,
      grid=(n // mt,),
      in_specs=[
          pl.BlockSpec((NC, mt, fin), lambda i: (0, i, 0)),
          pl.BlockSpec((mt, fin), lambda i: (i, 0)),
          pl.BlockSpec((mt, 128), lambda i: (i, 0)),
          pl.BlockSpec((1, fin), lambda i: (0, 0)),
          pl.BlockSpec((fin, fout), lambda i: (0, 0)),
      ],
      out_specs=pl.BlockSpec((mt, fout), lambda i: (i, 0)),
      out_shape=jax.ShapeDtypeStruct((n, fout), F32),
      compiler_params=pltpu.CompilerParams(
          dimension_semantics=("arbitrary",)),
  )(u, hp, cnt, b, w)


def _k_head(u, hp, cnt, b, fcw_row, mt=1000):
  n, fin = hp.shape

  def body(u_ref, h_ref, c_ref, b_ref, w_ref, o_ref):
    i = pl.program_id(0)

    @pl.when(i == 0)
    def _():
      o_ref[...] = jnp.zeros_like(o_ref)

    norm = lax.rsqrt(c_ref[:, 0:1] + 1.0)
    a = u_ref[0] + u_ref[1] + h_ref[...]
    a = jax.nn.relu(norm * a + b_ref[...])
    o_ref[...] += jnp.sum(a * w_ref[...], keepdims=True)

  return pl.pallas_call(
      body,
      grid=(n // mt,),
      in_specs=[
          pl.BlockSpec((NC, mt, fin), lambda i: (0, i, 0)),
          pl.BlockSpec((mt, fin), lambda i: (i, 0)),
          pl.BlockSpec((mt, 128), lambda i: (i, 0)),
          pl.BlockSpec((1, fin), lambda i: (0, 0)),
          pl.BlockSpec((1, fin), lambda i: (0, 0)),
      ],
      out_specs=pl.BlockSpec((1, 1), lambda i: (0, 0)),
      out_shape=jax.ShapeDtypeStruct((1, 1), F32),
      compiler_params=pltpu.CompilerParams(
          dimension_semantics=("arbitrary",)),
  )(u, hp, cnt, b, fcw_row)


def kernel(x, edge_index, W1, b1, W2, b2, W3, b3, W4, b4, fcW, fcb):
  n = x.shape[0]
  e = edge_index.shape[1]
  ep = _edge_pad(e)
  pad = ep - e

  src = edge_index[0]
  dst = edge_index[1]
  if pad:
    fill = jnp.arange(pad, dtype=jnp.int32)
    src = jnp.concatenate([src, fill % n])
    dst = jnp.concatenate([dst, n + (fill % PAD_ROWS)])

  npad = _node_pad(n)
  z128 = jnp.zeros((npad, 128), F32)
  ones = jnp.ones((KB, 128), F32)

  W4p = jnp.pad(W4, ((0, 0), (0, 64)))
  b4p = jnp.pad(b4, (0, 64))
  fcWp = jnp.pad(fcW[:, 0], (0, 64))

  cnt2 = _make_deg_kernel(npad, ep)(dst, ones, z128)

  h1, cnt = _k1(x, W1, cnt2)
  u1 = _make_agg_kernel(npad, ep, h1.shape[1])(h1, src, dst, z128)
  h2 = _k_mid(u1, h1, cnt, b1.reshape(1, -1), W2)
  u2 = _make_agg_kernel(npad, ep, h2.shape[1])(h2, src, dst, z128)
  h3 = _k_mid(u2, h2, cnt, b2.reshape(1, -1), W3)
  u3 = _make_agg_kernel(npad, ep, h3.shape[1])(h3, src, dst, z128)
  h4 = _k_mid(u3, h3, cnt, b3.reshape(1, -1), W4p)
  u4 = _make_agg_kernel(npad, ep, h4.shape[1])(h4, src, dst, z128)
  s = _k_head(u4, h4, cnt, b4p.reshape(1, -1), fcWp.reshape(1, -1))

  return jax.nn.sigmoid(s / n + fcb)


# speedup vs baseline: 1.1548x; 1.1548x over previous
import functools

import jax
import jax.numpy as jnp
from jax import lax
from jax.experimental import pallas as pl
from jax.experimental.pallas import tpu as pltpu
from jax.experimental.pallas import tpu_sc as plsc

F32 = jnp.float32
NC = 2
NS = 16
KB = 128
PAD_ROWS = 32


def _edge_pad(e):
  per_tile = -(-e // (NC * NS * 4 * KB)) * 4 * KB
  return per_tile * NC * NS


def _node_pad(n):
  npad = -(-n // 128) * 128
  if npad - n < PAD_ROWS:
    npad += 128
  return npad


def _make_deg_kernel(npad, ep):
  et = ep // (NC * NS)
  nb = et // KB
  ng = nb // 4
  rpt = npad // NS

  mesh = plsc.VectorSubcoreMesh(
      core_axis_name="c", subcore_axis_name="s", num_cores=NC, num_subcores=NS)

  @functools.partial(
      pl.kernel,
      out_type=jax.ShapeDtypeStruct((NC, npad, 128), F32),
      mesh=mesh,
      scratch_types=[
          pltpu.VMEM((et,), jnp.int32),
          pltpu.VMEM((KB,), jnp.int32),
          pltpu.VMEM((KB,), jnp.int32),
          pltpu.VMEM((KB,), jnp.int32),
          pltpu.VMEM((KB,), jnp.int32),
          pltpu.VMEM((KB, 128), F32),
          pltpu.VMEM_SHARED((npad, 128), F32),
          pltpu.SemaphoreType.DMA,
          pltpu.SemaphoreType.DMA,
          pltpu.SemaphoreType.DMA,
          pltpu.SemaphoreType.DMA,
      ],
  )
  def deg_kernel(dst_hbm, ones_hbm, zeros_hbm, out_hbm, dst_t, sidx0, sidx1,
                 sidx2, sidx3, ones_t, acc, sem0, sem1, sem2, sem3):
    ci = lax.axis_index("c")
    si = lax.axis_index("s")
    t = ci * NS + si
    r0 = si * rpt
    sidx = (sidx0, sidx1, sidx2, sidx3)
    sems = (sem0, sem1, sem2, sem3)
    pltpu.sync_copy(dst_hbm.at[pl.ds(t * et, et)], dst_t)
    pltpu.sync_copy(ones_hbm, ones_t)
    pltpu.sync_copy(zeros_hbm.at[pl.ds(r0, rpt)], acc.at[pl.ds(r0, rpt)])
    plsc.subcore_barrier()

    def quad(g, carry):
      @pl.when(g > 0)
      def _():
        for i in range(4):
          pltpu.make_async_copy(ones_t, acc.at[sidx[i]], sems[i]).wait()

      for i in range(4):
        base = pl.multiple_of((4 * g + i) * KB, KB)
        for j in range(KB // 16):
          sidx[i][pl.ds(j * 16, 16)] = dst_t[pl.ds(base + j * 16, 16)]
        pltpu.async_copy(ones_t, acc.at[sidx[i]], sems[i], add=True)
      return carry

    lax.fori_loop(0, ng, quad, 0)
    for i in range(4):
      pltpu.make_async_copy(ones_t, acc.at[sidx[i]], sems[i]).wait()
    plsc.subcore_barrier()
    pltpu.sync_copy(acc.at[pl.ds(r0, rpt)],
                    out_hbm.at[ci, pl.ds(r0, rpt)])

  return deg_kernel


def _make_agg_kernel(npad, ep, f):
  fc = min(f, 128)
  cb = f // fc
  et = ep // (NC * NS)
  nb = et // KB
  rpt = npad // NS

  mesh = plsc.VectorSubcoreMesh(
      core_axis_name="c", subcore_axis_name="s", num_cores=NC, num_subcores=NS)

  @functools.partial(
      pl.kernel,
      out_type=jax.ShapeDtypeStruct((NC, npad, f), F32),
      mesh=mesh,
      scratch_types=[
          pltpu.VMEM((et,), jnp.int32),
          pltpu.VMEM((et,), jnp.int32),
          pltpu.VMEM((KB,), jnp.int32),
          pltpu.VMEM((KB,), jnp.int32),
          pltpu.VMEM((KB, fc), F32),
          pltpu.VMEM((KB, fc), F32),
          pltpu.VMEM_SHARED((npad, fc), F32),
          pltpu.SemaphoreType.DMA,
          pltpu.SemaphoreType.DMA,
          pltpu.SemaphoreType.DMA,
          pltpu.SemaphoreType.DMA,
      ],
  )
  def agg_kernel(table_hbm, src_hbm, dst_hbm, zeros_hbm, u_hbm, src_t, dst_t,
                 sidx_a, sidx_b, gbuf_a, gbuf_b, acc, gsem_a, gsem_b, ssem_a,
                 ssem_b):
    ci = lax.axis_index("c")
    si = lax.axis_index("s")
    t = ci * NS + si
    r0 = si * rpt
    pltpu.sync_copy(src_hbm.at[pl.ds(t * et, et)], src_t)
    pltpu.sync_copy(dst_hbm.at[pl.ds(t * et, et)], dst_t)
    ng = nb // 2

    for c in range(cb):
      pltpu.sync_copy(zeros_hbm.at[pl.ds(r0, rpt)],
                      acc.at[pl.ds(r0, rpt)])
      plsc.subcore_barrier()

      def gather_src(base):
        gi = src_t.at[pl.ds(base, KB)]
        if cb > 1:
          return table_hbm.at[gi, pl.ds(c * fc, fc)]
        return table_hbm.at[gi]

      def gather_start(base, gbuf, gsem):
        return pltpu.async_copy(gather_src(base), gbuf, gsem)

      def build_sidx(base, sidx):
        for j in range(KB // 16):
          sidx[pl.ds(j * 16, 16)] = dst_t[pl.ds(base + j * 16, 16)]

      def batch_base(g):
        return pl.multiple_of(g * KB, KB)

      def pair(g, carry):
        b0 = batch_base(2 * g)
        b1 = batch_base(2 * g + 1)

        @pl.when(g == 0)
        def _():
          build_sidx(b0, sidx_a)
          gather_start(b0, gbuf_a, gsem_a)

        @pl.when(g > 0)
        def _():
          pltpu.make_async_copy(gbuf_b, acc.at[sidx_b], ssem_b).wait()

        build_sidx(b1, sidx_b)
        gather_start(b1, gbuf_b, gsem_b)

        pltpu.make_async_copy(gather_src(b0), gbuf_a, gsem_a).wait()
        pltpu.async_copy(gbuf_a, acc.at[sidx_a], ssem_a, add=True)
        pltpu.make_async_copy(gbuf_a, acc.at[sidx_a], ssem_a).wait()

        @pl.when(g < ng - 1)
        def _():
          b2 = batch_base(2 * g + 2)
          build_sidx(b2, sidx_a)
          gather_start(b2, gbuf_a, gsem_a)

        pltpu.make_async_copy(gather_src(b1), gbuf_b, gsem_b).wait()
        pltpu.async_copy(gbuf_b, acc.at[sidx_b], ssem_b, add=True)
        return carry

      lax.fori_loop(0, ng, pair, 0)
      pltpu.make_async_copy(gbuf_b, acc.at[sidx_b], ssem_b).wait()
      plsc.subcore_barrier()
      pltpu.sync_copy(acc.at[pl.ds(r0, rpt)],
                      u_hbm.at[ci, pl.ds(r0, rpt), pl.ds(c * fc, fc)])

  return agg_kernel


def _k1(x, w1, cnt2, mt=1000):
  n, kdim = x.shape
  fout = w1.shape[1]
  npad = cnt2.shape[1]

  def body(x_ref, w_ref, c_ref, o_ref, cs_ref):
    h = jnp.dot(x_ref[...], w_ref[...], preferred_element_type=F32)
    cs = c_ref[0] + c_ref[1]
    cs_ref[...] = cs
    norm = lax.rsqrt(cs[:, 0:1] + 1.0)
    o_ref[...] = h * norm

  return pl.pallas_call(
      body,
      grid=(n // mt,),
      in_specs=[
          pl.BlockSpec((mt, kdim), lambda i: (i, 0)),
          pl.BlockSpec((kdim, fout), lambda i: (0, 0)),
          pl.BlockSpec((NC, mt, 128), lambda i: (0, i, 0)),
      ],
      out_specs=[
          pl.BlockSpec((mt, fout), lambda i: (i, 0)),
          pl.BlockSpec((mt, 128), lambda i: (i, 0)),
      ],
      out_shape=[
          jax.ShapeDtypeStruct((n, fout), F32),
          jax.ShapeDtypeStruct((npad, 128), F32),
      ],
      compiler_params=pltpu.CompilerParams(
          dimension_semantics=("arbitrary",)),
  )(x, w1, cnt2)


def _k_mid(u, hp, cnt, b, w, mt=1000):
  n, fin = hp.shape
  fout = w.shape[1]

  def body(u_ref, h_ref, c_ref, b_ref, w_ref, o_ref):
    norm = lax.rsqrt(c_ref[:, 0:1] + 1.0)
    a = u_ref[0] + u_ref[1] + h_ref[...]
    a = jax.nn.relu(norm * a + b_ref[...])
    o_ref[...] = jnp.dot(a, w_ref[...], preferred_element_type=F32) * norm

  return pl.pallas_call(
      body,
      grid=(n // mt,),
      in_specs=[
          pl.BlockSpec((NC, mt, fin), lambda i: (0, i, 0)),
          pl.BlockSpec((mt, fin), lambda i: (i, 0)),
          pl.BlockSpec((mt, 128), lambda i: (i, 0)),
          pl.BlockSpec((1, fin), lambda i: (0, 0)),
          pl.BlockSpec((fin, fout), lambda i: (0, 0)),
      ],
      out_specs=pl.BlockSpec((mt, fout), lambda i: (i, 0)),
      out_shape=jax.ShapeDtypeStruct((n, fout), F32),
      compiler_params=pltpu.CompilerParams(
          dimension_semantics=("arbitrary",)),
  )(u, hp, cnt, b, w)


def _k_head(u, hp, cnt, b, fcw_row, mt=1000):
  n, fin = hp.shape

  def body(u_ref, h_ref, c_ref, b_ref, w_ref, o_ref):
    i = pl.program_id(0)

    @pl.when(i == 0)
    def _():
      o_ref[...] = jnp.zeros_like(o_ref)

    norm = lax.rsqrt(c_ref[:, 0:1] + 1.0)
    a = u_ref[0] + u_ref[1] + h_ref[...]
    a = jax.nn.relu(norm * a + b_ref[...])
    o_ref[...] += jnp.sum(a * w_ref[...], keepdims=True)

  return pl.pallas_call(
      body,
      grid=(n // mt,),
      in_specs=[
          pl.BlockSpec((NC, mt, fin), lambda i: (0, i, 0)),
          pl.BlockSpec((mt, fin), lambda i: (i, 0)),
          pl.BlockSpec((mt, 128), lambda i: (i, 0)),
          pl.BlockSpec((1, fin), lambda i: (0, 0)),
          pl.BlockSpec((1, fin), lambda i: (0, 0)),
      ],
      out_specs=pl.BlockSpec((1, 1), lambda i: (0, 0)),
      out_shape=jax.ShapeDtypeStruct((1, 1), F32),
      compiler_params=pltpu.CompilerParams(
          dimension_semantics=("arbitrary",)),
  )(u, hp, cnt, b, fcw_row)


def kernel(x, edge_index, W1, b1, W2, b2, W3, b3, W4, b4, fcW, fcb):
  n = x.shape[0]
  e = edge_index.shape[1]
  ep = _edge_pad(e)
  pad = ep - e

  src = edge_index[0]
  dst = edge_index[1]
  if pad:
    fill = jnp.arange(pad, dtype=jnp.int32)
    src = jnp.concatenate([src, fill % n])
    dst = jnp.concatenate([dst, n + (fill % PAD_ROWS)])

  npad = _node_pad(n)
  z128 = jnp.zeros((npad, 128), F32)
  ones = jnp.ones((KB, 128), F32)

  W4p = jnp.pad(W4, ((0, 0), (0, 64)))
  b4p = jnp.pad(b4, (0, 64))
  fcWp = jnp.pad(fcW[:, 0], (0, 64))

  cnt2 = _make_deg_kernel(npad, ep)(dst, ones, z128)

  h1, cnt = _k1(x, W1, cnt2)
  u1 = _make_agg_kernel(npad, ep, h1.shape[1])(h1, src, dst, z128)
  h2 = _k_mid(u1, h1, cnt, b1.reshape(1, -1), W2)
  u2 = _make_agg_kernel(npad, ep, h2.shape[1])(h2, src, dst, z128)
  h3 = _k_mid(u2, h2, cnt, b2.reshape(1, -1), W3)
  u3 = _make_agg_kernel(npad, ep, h3.shape[1])(h3, src, dst, z128)
  h4 = _k_mid(u3, h3, cnt, b3.reshape(1, -1), W4p)
  u4 = _make_agg_kernel(npad, ep, h4.shape[1])(h4, src, dst, z128)
  s = _k_head(u4, h4, cnt, b4p.reshape(1, -1), fcWp.reshape(1, -1))

  return jax.nn.sigmoid(s / n + fcb)


# speedup vs baseline: 1.1551x; 1.0003x over previous
import functools

import jax
import jax.numpy as jnp
from jax import lax
from jax.experimental import pallas as pl
from jax.experimental.pallas import tpu as pltpu
from jax.experimental.pallas import tpu_sc as plsc

F32 = jnp.float32
NC = 2
NS = 16
KB = 128
PAD_ROWS = 32


def _edge_pad(e):
  per_tile = -(-e // (NC * NS * 4 * KB)) * 4 * KB
  return per_tile * NC * NS


def _node_pad(n):
  npad = -(-n // 128) * 128
  if npad - n < PAD_ROWS:
    npad += 128
  return npad


def _make_deg_kernel(npad, ep):
  et = ep // (NC * NS)
  nb = et // KB
  ng = nb // 4
  rpt = npad // NS

  mesh = plsc.VectorSubcoreMesh(
      core_axis_name="c", subcore_axis_name="s", num_cores=NC, num_subcores=NS)

  @functools.partial(
      pl.kernel,
      out_type=jax.ShapeDtypeStruct((NC, npad, 128), F32),
      mesh=mesh,
      scratch_types=[
          pltpu.VMEM((et,), jnp.int32),
          pltpu.VMEM((KB,), jnp.int32),
          pltpu.VMEM((KB,), jnp.int32),
          pltpu.VMEM((KB,), jnp.int32),
          pltpu.VMEM((KB,), jnp.int32),
          pltpu.VMEM((KB, 128), F32),
          pltpu.VMEM_SHARED((npad, 128), F32),
          pltpu.SemaphoreType.DMA,
          pltpu.SemaphoreType.DMA,
          pltpu.SemaphoreType.DMA,
          pltpu.SemaphoreType.DMA,
      ],
  )
  def deg_kernel(dst_hbm, ones_hbm, zeros_hbm, out_hbm, dst_t, sidx0, sidx1,
                 sidx2, sidx3, ones_t, acc, sem0, sem1, sem2, sem3):
    ci = lax.axis_index("c")
    si = lax.axis_index("s")
    t = ci * NS + si
    r0 = si * rpt
    sidx = (sidx0, sidx1, sidx2, sidx3)
    sems = (sem0, sem1, sem2, sem3)
    pltpu.sync_copy(dst_hbm.at[pl.ds(t * et, et)], dst_t)
    pltpu.sync_copy(ones_hbm, ones_t)
    pltpu.sync_copy(zeros_hbm.at[pl.ds(r0, rpt)], acc.at[pl.ds(r0, rpt)])
    plsc.subcore_barrier()

    def quad(g, carry):
      @pl.when(g > 0)
      def _():
        for i in range(4):
          pltpu.make_async_copy(ones_t, acc.at[sidx[i]], sems[i]).wait()

      for i in range(4):
        base = pl.multiple_of((4 * g + i) * KB, KB)
        for j in range(KB // 16):
          sidx[i][pl.ds(j * 16, 16)] = dst_t[pl.ds(base + j * 16, 16)]
        pltpu.async_copy(ones_t, acc.at[sidx[i]], sems[i], add=True)
      return carry

    lax.fori_loop(0, ng, quad, 0)
    for i in range(4):
      pltpu.make_async_copy(ones_t, acc.at[sidx[i]], sems[i]).wait()
    plsc.subcore_barrier()
    pltpu.sync_copy(acc.at[pl.ds(r0, rpt)],
                    out_hbm.at[ci, pl.ds(r0, rpt)])

  return deg_kernel


def _make_agg_kernel(npad, ep, f):
  fc = min(f, 128)
  cb = f // fc
  et = ep // (NC * NS)
  nb = et // KB
  rpt = npad // NS

  mesh = plsc.VectorSubcoreMesh(
      core_axis_name="c", subcore_axis_name="s", num_cores=NC, num_subcores=NS)

  @functools.partial(
      pl.kernel,
      out_type=jax.ShapeDtypeStruct((NC, npad, f), F32),
      mesh=mesh,
      scratch_types=[
          pltpu.VMEM((et,), jnp.int32),
          pltpu.VMEM((et,), jnp.int32),
          pltpu.VMEM((KB,), jnp.int32),
          pltpu.VMEM((KB,), jnp.int32),
          pltpu.VMEM((KB, fc), F32),
          pltpu.VMEM((KB, fc), F32),
          pltpu.VMEM_SHARED((npad, fc), F32),
          pltpu.SemaphoreType.DMA,
          pltpu.SemaphoreType.DMA,
          pltpu.SemaphoreType.DMA,
          pltpu.SemaphoreType.DMA,
      ],
  )
  def agg_kernel(table_hbm, src_hbm, dst_hbm, zeros_hbm, u_hbm, src_t, dst_t,
                 sidx_a, sidx_b, gbuf_a, gbuf_b, acc, gsem_a, gsem_b, ssem_a,
                 ssem_b):
    ci = lax.axis_index("c")
    si = lax.axis_index("s")
    t = ci * NS + si
    r0 = si * rpt
    pltpu.sync_copy(src_hbm.at[pl.ds(t * et, et)], src_t)
    pltpu.sync_copy(dst_hbm.at[pl.ds(t * et, et)], dst_t)
    ng = nb // 2

    for c in range(cb):
      pltpu.sync_copy(zeros_hbm.at[pl.ds(r0, rpt)],
                      acc.at[pl.ds(r0, rpt)])
      plsc.subcore_barrier()

      def gather_src(base, c=c):
        gi = src_t.at[pl.ds(base, KB)]
        if cb > 1:
          return table_hbm.at[gi, pl.ds(c * fc, fc)]
        return table_hbm.at[gi]

      def gather_start(base, gbuf, gsem):
        return pltpu.async_copy(gather_src(base), gbuf, gsem)

      def build_sidx(base, sidx):
        for j in range(KB // 16):
          sidx[pl.ds(j * 16, 16)] = dst_t[pl.ds(base + j * 16, 16)]

      def batch_base(g):
        return pl.multiple_of(g * KB, KB)

      def pair(g, carry):
        b0 = batch_base(2 * g)
        b1 = batch_base(2 * g + 1)

        @pl.when(g == 0)
        def _():
          build_sidx(b0, sidx_a)
          gather_start(b0, gbuf_a, gsem_a)

        @pl.when(g > 0)
        def _():
          pltpu.make_async_copy(gbuf_b, acc.at[sidx_b], ssem_b).wait()

        build_sidx(b1, sidx_b)
        gather_start(b1, gbuf_b, gsem_b)

        pltpu.make_async_copy(gather_src(b0), gbuf_a, gsem_a).wait()
        pltpu.async_copy(gbuf_a, acc.at[sidx_a], ssem_a, add=True)
        pltpu.make_async_copy(gbuf_a, acc.at[sidx_a], ssem_a).wait()

        @pl.when(g < ng - 1)
        def _():
          b2 = batch_base(2 * g + 2)
          build_sidx(b2, sidx_a)
          gather_start(b2, gbuf_a, gsem_a)

        pltpu.make_async_copy(gather_src(b1), gbuf_b, gsem_b).wait()
        pltpu.async_copy(gbuf_b, acc.at[sidx_b], ssem_b, add=True)
        return carry

      lax.fori_loop(0, ng, pair, 0)
      pltpu.make_async_copy(gbuf_b, acc.at[sidx_b], ssem_b).wait()
      plsc.subcore_barrier()
      pltpu.sync_copy(acc.at[pl.ds(r0, rpt)],
                      u_hbm.at[ci, pl.ds(r0, rpt), pl.ds(c * fc, fc)])

  return agg_kernel


def _k1(x, w1, cnt2, mt=1000):
  n, kdim = x.shape
  fout = w1.shape[1]
  npad = cnt2.shape[1]

  def body(x_ref, w_ref, c_ref, o_ref, cs_ref):
    h = jnp.dot(x_ref[...], w_ref[...], preferred_element_type=F32)
    cs = c_ref[0] + c_ref[1]
    cs_ref[...] = cs
    norm = lax.rsqrt(cs[:, 0:1] + 1.0)
    o_ref[...] = h * norm

  return pl.pallas_call(
      body,
      grid=(n // mt,),
      in_specs=[
          pl.BlockSpec((mt, kdim), lambda i: (i, 0)),
          pl.BlockSpec((kdim, fout), lambda i: (0, 0)),
          pl.BlockSpec((NC, mt, 128), lambda i: (0, i, 0)),
      ],
      out_specs=[
          pl.BlockSpec((mt, fout), lambda i: (i, 0)),
          pl.BlockSpec((mt, 128), lambda i: (i, 0)),
      ],
      out_shape=[
          jax.ShapeDtypeStruct((n, fout), F32),
          jax.ShapeDtypeStruct((npad, 128), F32),
      ],
      compiler_params=pltpu.CompilerParams(
          dimension_semantics=("arbitrary",)),
  )(x, w1, cnt2)


def _k_mid(u, hp, cnt, b, w, mt=1000):
  n, fin = hp.shape
  fout = w.shape[1]
  partial = u.ndim == 3

  def body(u_ref, h_ref, c_ref, b_ref, w_ref, o_ref):
    norm = lax.rsqrt(c_ref[:, 0:1] + 1.0)
    if partial:
      a = u_ref[0] + u_ref[1] + h_ref[...]
    else:
      a = u_ref[...] + h_ref[...]
    a = jax.nn.relu(norm * a + b_ref[...])
    o_ref[...] = jnp.dot(a, w_ref[...], preferred_element_type=F32) * norm

  return pl.pallas_call(
      body,
      grid=(n // mt,),
      in_specs=[
          pl.BlockSpec((NC, mt, fin), lambda i: (0, i, 0)) if partial
          else pl.BlockSpec((mt, fin), lambda i: (i, 0)),
          pl.BlockSpec((mt, fin), lambda i: (i, 0)),
          pl.BlockSpec((mt, 128), lambda i: (i, 0)),
          pl.BlockSpec((1, fin), lambda i: (0, 0)),
          pl.BlockSpec((fin, fout), lambda i: (0, 0)),
      ],
      out_specs=pl.BlockSpec((mt, fout), lambda i: (i, 0)),
      out_shape=jax.ShapeDtypeStruct((n, fout), F32),
      compiler_params=pltpu.CompilerParams(
          dimension_semantics=("arbitrary",)),
  )(u, hp, cnt, b, w)


def _k_head(u, hp, cnt, b, fcw_row, mt=1000):
  n, fin = hp.shape
  partial = u.ndim == 3

  def body(u_ref, h_ref, c_ref, b_ref, w_ref, o_ref):
    i = pl.program_id(0)

    @pl.when(i == 0)
    def _():
      o_ref[...] = jnp.zeros_like(o_ref)

    norm = lax.rsqrt(c_ref[:, 0:1] + 1.0)
    if partial:
      a = u_ref[0] + u_ref[1] + h_ref[...]
    else:
      a = u_ref[...] + h_ref[...]
    a = jax.nn.relu(norm * a + b_ref[...])
    o_ref[...] += jnp.sum(a * w_ref[...], keepdims=True)

  return pl.pallas_call(
      body,
      grid=(n // mt,),
      in_specs=[
          pl.BlockSpec((NC, mt, fin), lambda i: (0, i, 0)) if partial
          else pl.BlockSpec((mt, fin), lambda i: (i, 0)),
          pl.BlockSpec((mt, fin), lambda i: (i, 0)),
          pl.BlockSpec((mt, 128), lambda i: (i, 0)),
          pl.BlockSpec((1, fin), lambda i: (0, 0)),
          pl.BlockSpec((1, fin), lambda i: (0, 0)),
      ],
      out_specs=pl.BlockSpec((1, 1), lambda i: (0, 0)),
      out_shape=jax.ShapeDtypeStruct((1, 1), F32),
      compiler_params=pltpu.CompilerParams(
          dimension_semantics=("arbitrary",)),
  )(u, hp, cnt, b, fcw_row)


def kernel(x, edge_index, W1, b1, W2, b2, W3, b3, W4, b4, fcW, fcb):
  n = x.shape[0]
  e = edge_index.shape[1]
  ep = _edge_pad(e)
  pad = ep - e

  src = edge_index[0]
  dst = edge_index[1]
  if pad:
    fill = jnp.arange(pad, dtype=jnp.int32)
    src = jnp.concatenate([src, fill % n])
    dst = jnp.concatenate([dst, n + (fill % PAD_ROWS)])

  npad = _node_pad(n)
  z128 = jnp.zeros((npad, 128), F32)
  ones = jnp.ones((KB, 128), F32)

  W4p = jnp.pad(W4, ((0, 0), (0, 64)))
  b4p = jnp.pad(b4, (0, 64))
  fcWp = jnp.pad(fcW[:, 0], (0, 64))

  cnt2 = _make_deg_kernel(npad, ep)(dst, ones, z128)

  h1, cnt = _k1(x, W1, cnt2)
  u1 = _make_agg_kernel(npad, ep, h1.shape[1])(h1, src, dst, z128)
  h2 = _k_mid(u1, h1, cnt, b1.reshape(1, -1), W2)
  u2 = _make_agg_kernel(npad, ep, h2.shape[1])(h2, src, dst, z128)
  h3 = _k_mid(u2, h2, cnt, b2.reshape(1, -1), W3)
  u3 = _make_agg_kernel(npad, ep, h3.shape[1])(h3, src, dst, z128)
  h4 = _k_mid(u3, h3, cnt, b3.reshape(1, -1), W4p)
  u4 = _make_agg_kernel(npad, ep, h4.shape[1])(h4, src, dst, z128)
  s = _k_head(u4, h4, cnt, b4p.reshape(1, -1), fcWp.reshape(1, -1))

  return jax.nn.sigmoid(s / n + fcb)
